# baseline (speedup 1.0000x reference)
import jax
import jax.numpy as jnp
from jax.experimental import pallas as pl

_EDGE_IN = 19
_LANES = 128
_CHUNK = _EDGE_IN * _LANES
_BLOCK_G = 125


def _edge_pred_kernel(ef_ref, g_ref, b_ref, o_ref):
    o_ref[0] = (
        jnp.dot(ef_ref[0], g_ref[...], preferred_element_type=jnp.float32)
        + b_ref[...]
    )


def kernel(x, edge_index, e, xbatch, bn_g0, bn_b0, W00, b00, W01, b01,
           root0, rb0, bn_g1, bn_b1, W10, b10, W11, b11, root1, rb1,
           bn_g2, bn_b2, W20, b20, W21, b21, root2, rb2, Wp, bp):
    e = e.reshape(-1, _EDGE_IN)
    n_edges = e.shape[0]
    n_out = Wp.shape[1]
    rows = n_edges // _LANES
    ef = e.reshape(rows, _CHUNK)

    t = jnp.arange(_CHUNK)
    c = jnp.arange(_LANES * n_out)
    mask = (t[:, None] // _EDGE_IN) == (c[None, :] // n_out)
    g_mat = jnp.tile(Wp, (_LANES, _LANES)) * mask
    bias = jnp.tile(bp, (_LANES,)).reshape(1, _LANES * n_out)

    block = _BLOCK_G if rows % _BLOCK_G == 0 else rows
    steps = rows // block
    ef3 = ef.reshape(steps, block, _CHUNK)
    out2 = pl.pallas_call(
        _edge_pred_kernel,
        grid=(steps,),
        in_specs=[
            pl.BlockSpec((1, block, _CHUNK), lambda i: (i, 0, 0)),
            pl.BlockSpec((_CHUNK, _LANES * n_out), lambda i: (0, 0)),
            pl.BlockSpec((1, _LANES * n_out), lambda i: (0, 0)),
        ],
        out_specs=pl.BlockSpec((1, block, _LANES * n_out), lambda i: (i, 0, 0)),
        out_shape=jax.ShapeDtypeStruct((steps, block, _LANES * n_out), jnp.float32),
    )(ef3, g_mat, bias)
    return out2.reshape(n_edges, n_out)


# speedup vs baseline: 28.1120x; 28.1120x over previous
import jax
import jax.numpy as jnp
from jax.experimental import pallas as pl

_EDGE_IN = 19
_BLOCK_C = 16000


def _edge_pred_kernel(w_ref, et_ref, b_ref, o_ref):
    o_ref[...] = (
        jnp.dot(w_ref[...], et_ref[...], preferred_element_type=jnp.float32)
        + b_ref[...]
    )


def kernel(x, edge_index, e, xbatch, bn_g0, bn_b0, W00, b00, W01, b01,
           root0, rb0, bn_g1, bn_b1, W10, b10, W11, b11, root1, rb1,
           bn_g2, bn_b2, W20, b20, W21, b21, root2, rb2, Wp, bp):
    e = e.reshape(-1, _EDGE_IN)
    n_edges = e.shape[0]
    n_out = Wp.shape[1]
    et = e.T
    wt = Wp.T
    bias = bp.reshape(n_out, 1)

    block = _BLOCK_C if n_edges % _BLOCK_C == 0 else n_edges
    out_t = pl.pallas_call(
        _edge_pred_kernel,
        grid=(n_edges // block,),
        in_specs=[
            pl.BlockSpec((n_out, _EDGE_IN), lambda i: (0, 0)),
            pl.BlockSpec((_EDGE_IN, block), lambda i: (0, i)),
            pl.BlockSpec((n_out, 1), lambda i: (0, 0)),
        ],
        out_specs=pl.BlockSpec((n_out, block), lambda i: (0, i)),
        out_shape=jax.ShapeDtypeStruct((n_out, n_edges), jnp.float32),
    )(wt, et, bias)
    return out_t.T


# speedup vs baseline: 36.9771x; 1.3153x over previous
import jax
import jax.numpy as jnp
from jax.experimental import pallas as pl
from jax.experimental.pallas import tpu as pltpu

_EDGE_IN = 19


def _edge_pred_kernel(w_ref, et_ref, b_ref, o_ref):
    o_ref[...] = (
        jnp.dot(w_ref[...], et_ref[...], preferred_element_type=jnp.float32)
        + b_ref[...]
    )


def kernel(x, edge_index, e, xbatch, bn_g0, bn_b0, W00, b00, W01, b01,
           root0, rb0, bn_g1, bn_b1, W10, b10, W11, b11, root1, rb1,
           bn_g2, bn_b2, W20, b20, W21, b21, root2, rb2, Wp, bp):
    e = e.reshape(-1, _EDGE_IN)
    n_edges = e.shape[0]
    n_out = Wp.shape[1]
    et = e.T
    wt = Wp.T
    bias = bp.reshape(n_out, 1)

    out_t = pl.pallas_call(
        _edge_pred_kernel,
        in_specs=[
            pl.BlockSpec(memory_space=pltpu.VMEM),
            pl.BlockSpec(memory_space=pltpu.VMEM),
            pl.BlockSpec(memory_space=pltpu.VMEM),
        ],
        out_specs=pl.BlockSpec(memory_space=pltpu.VMEM),
        out_shape=jax.ShapeDtypeStruct((n_out, n_edges), jnp.float32),
    )(wt, et, bias)
    return out_t.T
